# baseline (speedup 1.0000x reference)
import functools

import jax
import jax.numpy as jnp
from jax import lax
from jax.experimental import pallas as pl
from jax.experimental.pallas import tpu as pltpu
from jax.experimental.pallas import tpu_sc as plsc

_TB = 64
_NW = 32
_LANES = 16


def _make_sc_gather(n, v):
    per_w = n // _NW
    mesh = plsc.VectorSubcoreMesh(core_axis_name="c", subcore_axis_name="s")

    @functools.partial(
        pl.kernel,
        mesh=mesh,
        out_type=jax.ShapeDtypeStruct((n,), jnp.float32),
        scratch_types=[
            pltpu.VMEM((per_w,), jnp.int32),
            pltpu.VMEM((per_w,), jnp.int32),
            pltpu.VMEM((per_w,), jnp.float32),
            pltpu.SemaphoreType.DMA,
        ],
    )
    def sc_gather(preds_hbm, tgt_hbm, out_hbm, tgt_v, idx_v, g_v, sem):
        wid = lax.axis_index("s") * 2 + lax.axis_index("c")
        base = wid * per_w
        pltpu.sync_copy(tgt_hbm.at[pl.ds(base, per_w)], tgt_v)
        for j in range(per_w // _LANES):
            t = tgt_v[pl.ds(j * _LANES, _LANES)]
            row = base + j * _LANES + lax.iota(jnp.int32, _LANES)
            idx_v[pl.ds(j * _LANES, _LANES)] = row * v + t
        pltpu.async_copy(preds_hbm.at[idx_v], g_v, sem).wait()
        pltpu.sync_copy(g_v, out_hbm.at[pl.ds(base, per_w)])

    return sc_gather


def _loss_block_kernel(preds_ref, tgt_ref, reward_ref, g_ref, out_ref):
    i = pl.program_id(0)
    x = preds_ref[...]
    m = jnp.max(x, axis=1)
    s = jnp.sum(jnp.exp(x - m[:, None]), axis=1)
    tgt = tgt_ref[0, 0, :]
    sel = jnp.exp(g_ref[0, 0, :] - m) / s
    mask = (tgt > 0).astype(jnp.float32)
    partial = jnp.sum(sel * mask * reward_ref[0, 0, :])

    @pl.when(i == 0)
    def _init():
        out_ref[...] = jnp.zeros_like(out_ref)

    out_ref[...] += jnp.full(out_ref.shape, -partial, out_ref.dtype)


def kernel(preds, tgt, tgt_pos, reward):
    b, seq, v = preds.shape
    n = b * seq
    nt = n // _TB
    preds2 = preds.reshape(n, v)
    tgt_flat = tgt.reshape(n)
    tgt3 = tgt.reshape(nt, 1, _TB)
    reward3 = reward.reshape(nt, 1, _TB)

    g = _make_sc_gather(n, v)(preds.reshape(n * v), tgt_flat)
    g3 = g.reshape(nt, 1, _TB)

    out = pl.pallas_call(
        _loss_block_kernel,
        grid=(nt,),
        in_specs=[
            pl.BlockSpec((_TB, v), lambda i: (i, 0)),
            pl.BlockSpec((1, 1, _TB), lambda i: (i, 0, 0)),
            pl.BlockSpec((1, 1, _TB), lambda i: (i, 0, 0)),
            pl.BlockSpec((1, 1, _TB), lambda i: (i, 0, 0)),
        ],
        out_specs=pl.BlockSpec((1, 1), lambda i: (0, 0)),
        out_shape=jax.ShapeDtypeStruct((1, 1), jnp.float32),
    )(preds2, tgt3, reward3, g3)
    return out[0, 0]


# speedup vs baseline: 2.6936x; 2.6936x over previous
import jax
import jax.numpy as jnp
from jax.experimental import pallas as pl

_TB = 64


def _loss_block_kernel(preds_ref, tgt_ref, reward_ref, out_ref):
    i = pl.program_id(0)
    x = preds_ref[...]
    tb, v = x.shape
    tgt = tgt_ref[0, 0, :]
    cols = jax.lax.broadcasted_iota(jnp.int32, (tb, v), 1)
    neg = jnp.float32(-jnp.inf)
    g = jnp.max(jnp.where(cols == tgt[:, None], x, neg), axis=1)
    m = jnp.max(x, axis=1)
    s = jnp.sum(jnp.exp(x - m[:, None]), axis=1)
    sel = jnp.exp(g - m) / s
    mask = (tgt > 0).astype(jnp.float32)
    partial = jnp.sum(sel * mask * reward_ref[0, 0, :])

    @pl.when(i == 0)
    def _init():
        out_ref[...] = jnp.zeros_like(out_ref)

    out_ref[...] += jnp.full(out_ref.shape, -partial, out_ref.dtype)


def kernel(preds, tgt, tgt_pos, reward):
    b, seq, v = preds.shape
    n = b * seq
    nt = n // _TB
    preds2 = preds.reshape(n, v)
    tgt3 = tgt.reshape(nt, 1, _TB)
    reward3 = reward.reshape(nt, 1, _TB)

    out = pl.pallas_call(
        _loss_block_kernel,
        grid=(nt,),
        in_specs=[
            pl.BlockSpec((_TB, v), lambda i: (i, 0)),
            pl.BlockSpec((1, 1, _TB), lambda i: (i, 0, 0)),
            pl.BlockSpec((1, 1, _TB), lambda i: (i, 0, 0)),
        ],
        out_specs=pl.BlockSpec((1, 1), lambda i: (0, 0)),
        out_shape=jax.ShapeDtypeStruct((1, 1), jnp.float32),
    )(preds2, tgt3, reward3)
    return out[0, 0]
